# Initial kernel scaffold; baseline (speedup 1.0000x reference)
import jax
import jax.numpy as jnp
from jax.experimental import pallas as pl


def kernel(input_features_b_l, input_length_b, table, W1, b1, W2, b2):
    raise NotImplementedError("write your pallas kernel here")


# speedup vs baseline: 3.9860x; 3.9860x over previous
import functools
import jax
import jax.numpy as jnp
from jax import lax
from jax.experimental import pallas as pl
from jax.experimental.pallas import tpu as pltpu
from jax.experimental.pallas import tpu_sc as plsc

B, L, V, D, H, C = 4096, 50, 100000, 128, 128, 20

NC, NS = 2, 16
NW = NC * NS
BW = B // NW
NB = 4
NCHUNK = BW // NB
GOFF = (0, 104)
GLEN = (104, 96)


def _emb_sum_body(table_hbm, idx_hbm, out_hbm, idx_v, rows_v, out_v, sem):
    wid = lax.axis_index("s") * NC + lax.axis_index("c")
    flat_base = wid * (BW * L)
    row_base = wid * BW

    def chunk(i, carry):
        flat0 = flat_base + i * (NB * L)
        row0 = row_base + i * NB
        pltpu.sync_copy(idx_hbm.at[pl.ds(flat0, NB * L)], idx_v)
        for off, n in zip(GOFF, GLEN):
            pltpu.async_copy(
                table_hbm.at[idx_v.at[pl.ds(off, n)]],
                rows_v.at[pl.ds(off, n)],
                sem,
            ).wait()
        for b in range(NB):
            for k in range(D // 16):
                sl = pl.ds(k * 16, 16)
                acc = rows_v[b * L, sl]
                for l in range(1, L):
                    acc = acc + rows_v[b * L + l, sl]
                out_v[b, sl] = acc
        pltpu.sync_copy(out_v, out_hbm.at[pl.ds(row0, NB)])
        return carry

    lax.fori_loop(0, NCHUNK, chunk, 0)


@functools.partial(
    pl.kernel,
    mesh=plsc.VectorSubcoreMesh(core_axis_name="c", subcore_axis_name="s"),
    out_type=jax.ShapeDtypeStruct((B, D), jnp.float32),
    scratch_types=[
        pltpu.VMEM((NB * L,), jnp.int32),
        pltpu.VMEM((NB * L, D), jnp.float32),
        pltpu.VMEM((NB, D), jnp.float32),
        pltpu.SemaphoreType.DMA,
    ],
)
def _emb_sum(table_hbm, idx_hbm, out_hbm, idx_v, rows_v, out_v, sem):
    _emb_sum_body(table_hbm, idx_hbm, out_hbm, idx_v, rows_v, out_v, sem)


def _mlp_body(sums_ref, idx_ref, w1_ref, b1_ref, w2_ref, b2_ref, out_ref):
    s = sums_ref[...]
    idxb = idx_ref[...]
    cnt = jnp.sum((idxb != 0).astype(jnp.float32), axis=1, keepdims=True)
    mean = s * (1.0 / jnp.maximum(cnt, 1.0))
    h = lax.dot_general(
        mean, w1_ref[...], (((1,), (1,)), ((), ())),
        preferred_element_type=jnp.float32,
    ) + b1_ref[...]
    h = jnp.maximum(h, 0.0)
    out = lax.dot_general(
        h, w2_ref[...], (((1,), (1,)), ((), ())),
        preferred_element_type=jnp.float32,
    ) + b2_ref[...]
    out_ref[...] = out


def kernel(input_features_b_l, input_length_b, table, W1, b1, W2, b2):
    del input_length_b
    idx = input_features_b_l.astype(jnp.int32)
    idx_flat = idx.reshape(-1)
    sums = _emb_sum(table, idx_flat)
    out = pl.pallas_call(
        _mlp_body,
        out_shape=jax.ShapeDtypeStruct((B, C), jnp.float32),
    )(sums, idx, W1, b1.reshape(1, H), W2, b2.reshape(1, C))
    return out


# speedup vs baseline: 5.2141x; 1.3081x over previous
import functools
import jax
import jax.numpy as jnp
from jax import lax
from jax.experimental import pallas as pl
from jax.experimental.pallas import tpu as pltpu
from jax.experimental.pallas import tpu_sc as plsc

B, L, V, D, H, C = 4096, 50, 100000, 128, 128, 20

NC, NS = 2, 16
NW = NC * NS
BW = B // NW
NB = 4
NCHUNK = BW // NB
GOFF = (0, 104)
GLEN = (104, 96)


def _emb_sum_body(table_hbm, idx_hbm, out_hbm, idx_v, rows0, rows1, out_all,
                  sem0, sem1):
    wid = lax.axis_index("s") * NC + lax.axis_index("c")
    flat_base = wid * (BW * L)
    row_base = wid * BW

    pltpu.sync_copy(idx_hbm.at[pl.ds(flat_base, BW * L)], idx_v)

    rows = (rows0, rows1)
    sems = (sem0, sem1)

    def fire(i, p):
        for off, n in zip(GOFF, GLEN):
            pltpu.async_copy(
                table_hbm.at[idx_v.at[pl.ds(i * (NB * L) + off, n)]],
                rows[p].at[pl.ds(off, n)],
                sems[p],
            )

    def drain(p):
        pltpu.make_async_copy(
            table_hbm.at[pl.ds(0, NB * L)], rows[p], sems[p]
        ).wait()

    def reduce(i, p):
        rbuf = rows[p]
        for b in range(NB):
            base = b * L
            accs = [rbuf[base, pl.ds(k * 16, 16)] for k in range(D // 16)]
            for l in range(1, L):
                for k in range(D // 16):
                    accs[k] = accs[k] + rbuf[base + l, pl.ds(k * 16, 16)]
            row = i * NB + b
            for k in range(D // 16):
                out_all[row, pl.ds(k * 16, 16)] = accs[k]

    fire(0, 0)

    def pair(j, carry):
        i0 = 2 * j
        fire(i0 + 1, 1)
        drain(0)
        reduce(i0, 0)

        @pl.when(j < (NCHUNK // 2) - 1)
        def _():
            fire(i0 + 2, 0)

        drain(1)
        reduce(i0 + 1, 1)
        return carry

    lax.fori_loop(0, NCHUNK // 2, pair, 0)
    pltpu.sync_copy(out_all, out_hbm.at[pl.ds(row_base, BW)])


@functools.partial(
    pl.kernel,
    mesh=plsc.VectorSubcoreMesh(core_axis_name="c", subcore_axis_name="s"),
    out_type=jax.ShapeDtypeStruct((B, D), jnp.float32),
    scratch_types=[
        pltpu.VMEM((BW * L,), jnp.int32),
        pltpu.VMEM((NB * L, D), jnp.float32),
        pltpu.VMEM((NB * L, D), jnp.float32),
        pltpu.VMEM((BW, D), jnp.float32),
        pltpu.SemaphoreType.DMA,
        pltpu.SemaphoreType.DMA,
    ],
)
def _emb_sum(table_hbm, idx_hbm, out_hbm, idx_v, rows0, rows1, out_all,
             sem0, sem1):
    _emb_sum_body(table_hbm, idx_hbm, out_hbm, idx_v, rows0, rows1, out_all,
                  sem0, sem1)


def _mlp_body(sums_ref, idx_ref, w1_ref, b1_ref, w2_ref, b2_ref, out_ref):
    s = sums_ref[...]
    idxb = idx_ref[...]
    cnt = jnp.sum((idxb != 0).astype(jnp.float32), axis=1, keepdims=True)
    mean = s * (1.0 / jnp.maximum(cnt, 1.0))
    h = lax.dot_general(
        mean, w1_ref[...], (((1,), (1,)), ((), ())),
        preferred_element_type=jnp.float32,
    ) + b1_ref[...]
    h = jnp.maximum(h, 0.0)
    out = lax.dot_general(
        h, w2_ref[...], (((1,), (1,)), ((), ())),
        preferred_element_type=jnp.float32,
    ) + b2_ref[...]
    out_ref[...] = out


def kernel(input_features_b_l, input_length_b, table, W1, b1, W2, b2):
    del input_length_b
    idx = input_features_b_l.astype(jnp.int32)
    idx_flat = idx.reshape(-1)
    sums = _emb_sum(table, idx_flat)
    out = pl.pallas_call(
        _mlp_body,
        out_shape=jax.ShapeDtypeStruct((B, C), jnp.float32),
    )(sums, idx, W1, b1.reshape(1, H), W2, b2.reshape(1, C))
    return out


# speedup vs baseline: 7.1967x; 1.3802x over previous
import functools
import jax
import jax.numpy as jnp
from jax import lax
from jax.experimental import pallas as pl
from jax.experimental.pallas import tpu as pltpu
from jax.experimental.pallas import tpu_sc as plsc

B, L, V, D, H, C = 4096, 50, 100000, 128, 128, 20

NC, NS = 2, 16
NW = NC * NS
BW = B // NW
NB = 4
NCHUNK = BW // NB
GOFF = (0, 104)
GLEN = (104, 96)


def _emb_sum_body(table_hbm, idx_hbm, out_hbm, idx_v, rows0, rows1, out_all,
                  sem0, sem1):
    wid = lax.axis_index("s") * NC + lax.axis_index("c")
    flat_base = wid * (BW * L)
    row_base = wid * BW

    pltpu.sync_copy(idx_hbm.at[pl.ds(flat_base, BW * L)], idx_v)

    rows = (rows0, rows1)
    sems = (sem0, sem1)

    def fire(i, p):
        for off, n in zip(GOFF, GLEN):
            pltpu.async_copy(
                table_hbm.at[idx_v.at[pl.ds(i * (NB * L) + off, n)]],
                rows[p].at[pl.ds(off, n)],
                sems[p],
            )

    def drain(p):
        pltpu.make_async_copy(
            table_hbm.at[pl.ds(0, NB * L)], rows[p], sems[p]
        ).wait()

    def reduce(i, p):
        rbuf = rows[p]
        G = 10
        for b in range(NB):
            base = b * L
            row = i * NB + b
            for g in range(L // G):
                accs = [rbuf[base + g * G, pl.ds(k * 16, 16)]
                        for k in range(D // 16)]
                for l in range(1, G):
                    for k in range(D // 16):
                        accs[k] = accs[k] + rbuf[base + g * G + l,
                                                 pl.ds(k * 16, 16)]
                for k in range(D // 16):
                    sl = pl.ds(k * 16, 16)
                    if g == 0:
                        out_all[row, sl] = accs[k]
                    else:
                        plsc.addupdate(out_all.at[row, sl], accs[k])

    fire(0, 0)

    def pair(j, carry):
        i0 = 2 * j
        fire(i0 + 1, 1)
        drain(0)
        reduce(i0, 0)

        @pl.when(j < (NCHUNK // 2) - 1)
        def _():
            fire(i0 + 2, 0)

        drain(1)
        reduce(i0 + 1, 1)
        return carry

    lax.fori_loop(0, NCHUNK // 2, pair, 0)
    pltpu.sync_copy(out_all, out_hbm.at[pl.ds(row_base, BW)])


@functools.partial(
    pl.kernel,
    mesh=plsc.VectorSubcoreMesh(core_axis_name="c", subcore_axis_name="s"),
    out_type=jax.ShapeDtypeStruct((B, D), jnp.float32),
    scratch_types=[
        pltpu.VMEM((BW * L,), jnp.int32),
        pltpu.VMEM((NB * L, D), jnp.float32),
        pltpu.VMEM((NB * L, D), jnp.float32),
        pltpu.VMEM((BW, D), jnp.float32),
        pltpu.SemaphoreType.DMA,
        pltpu.SemaphoreType.DMA,
    ],
)
def _emb_sum(table_hbm, idx_hbm, out_hbm, idx_v, rows0, rows1, out_all,
             sem0, sem1):
    _emb_sum_body(table_hbm, idx_hbm, out_hbm, idx_v, rows0, rows1, out_all,
                  sem0, sem1)


def _mlp_body(sums_ref, idx_ref, w1_ref, b1_ref, w2_ref, b2_ref, out_ref):
    s = sums_ref[...]
    idxb = idx_ref[...]
    cnt = jnp.sum((idxb != 0).astype(jnp.float32), axis=1, keepdims=True)
    mean = s * (1.0 / jnp.maximum(cnt, 1.0))
    h = lax.dot_general(
        mean, w1_ref[...], (((1,), (1,)), ((), ())),
        preferred_element_type=jnp.float32,
    ) + b1_ref[...]
    h = jnp.maximum(h, 0.0)
    out = lax.dot_general(
        h, w2_ref[...], (((1,), (1,)), ((), ())),
        preferred_element_type=jnp.float32,
    ) + b2_ref[...]
    out_ref[...] = out


def kernel(input_features_b_l, input_length_b, table, W1, b1, W2, b2):
    del input_length_b
    idx = input_features_b_l.astype(jnp.int32)
    idx_flat = idx.reshape(-1)
    sums = _emb_sum(table, idx_flat)
    out = pl.pallas_call(
        _mlp_body,
        out_shape=jax.ShapeDtypeStruct((B, C), jnp.float32),
    )(sums, idx, W1, b1.reshape(1, H), W2, b2.reshape(1, C))
    return out


# speedup vs baseline: 9.2961x; 1.2917x over previous
import functools
import jax
import jax.numpy as jnp
from jax import lax
from jax.experimental import pallas as pl
from jax.experimental.pallas import tpu as pltpu
from jax.experimental.pallas import tpu_sc as plsc

B, L, V, D, H, C = 4096, 50, 100000, 128, 128, 20

NC, NS = 2, 16
NW = NC * NS
BW = B // NW
NB = 4
NCHUNK = BW // NB
GOFF = (0, 104)
GLEN = (104, 96)
NR = NB * L + 8
SOFF = (0, 112)
SLEN = (112, 96)


def _emb_sum_body(table_hbm, idx_hbm, out_hbm, idx_v, rows0, rows1, out_all,
                  zbuf, sacc, didx0, didx1, sem0, sem1):
    didx = (didx0, didx1)
    sid = lax.axis_index("s")
    wid = sid * NC + lax.axis_index("c")
    flat_base = wid * (BW * L)
    row_base = wid * BW

    pltpu.sync_copy(idx_hbm.at[pl.ds(flat_base, BW * L)], idx_v)

    lanes = lax.iota(jnp.int32, 16)
    for r, (off, n) in enumerate(zip(SOFF, SLEN)):
        dref = didx[r]
        for g in range(n // 16):
            j0 = off + g * 16
            vals = jnp.minimum(((lanes + j0) * 1311) >> 16, NB - 1)
            dref[pl.ds(g * 16, 16)] = sid * NB + vals

    zero = jnp.zeros((16,), jnp.float32)
    for b in range(NB):
        for k in range(D // 16):
            zbuf[b, pl.ds(k * 16, 16)] = zero
    for rbuf in (rows0, rows1):
        for j in range(NB * L, NR):
            for k in range(D // 16):
                rbuf[j, pl.ds(k * 16, 16)] = zero

    rows = (rows0, rows1)
    sems = (sem0, sem1)

    def fire(i, p):
        for off, n in zip(GOFF, GLEN):
            pltpu.async_copy(
                table_hbm.at[idx_v.at[pl.ds(i * (NB * L) + off, n)]],
                rows[p].at[pl.ds(off, n)],
                sems[p],
            )

    def drain(p):
        pltpu.make_async_copy(
            table_hbm.at[pl.ds(0, NB * L)], rows[p].at[pl.ds(0, NB * L)],
            sems[p],
        ).wait()

    def reduce(i, p):
        myacc = sacc.at[pl.ds(sid * NB, NB)]
        pltpu.sync_copy(zbuf, myacc)
        for s, (off, n) in enumerate(zip(SOFF, SLEN)):
            pltpu.sync_copy(
                rows[p].at[pl.ds(off, n)], sacc.at[didx[s]], add=True
            )
        pltpu.sync_copy(myacc, out_all.at[pl.ds(i * NB, NB)])

    fire(0, 0)

    def pair(j, carry):
        i0 = 2 * j
        fire(i0 + 1, 1)
        drain(0)
        reduce(i0, 0)

        @pl.when(j < (NCHUNK // 2) - 1)
        def _():
            fire(i0 + 2, 0)

        drain(1)
        reduce(i0 + 1, 1)
        return carry

    lax.fori_loop(0, NCHUNK // 2, pair, 0)
    pltpu.sync_copy(out_all, out_hbm.at[pl.ds(row_base, BW)])


@functools.partial(
    pl.kernel,
    mesh=plsc.VectorSubcoreMesh(core_axis_name="c", subcore_axis_name="s"),
    out_type=jax.ShapeDtypeStruct((B, D), jnp.float32),
    scratch_types=[
        pltpu.VMEM((BW * L,), jnp.int32),
        pltpu.VMEM((NR, D), jnp.float32),
        pltpu.VMEM((NR, D), jnp.float32),
        pltpu.VMEM((BW, D), jnp.float32),
        pltpu.VMEM((NB, D), jnp.float32),
        pltpu.VMEM_SHARED((NS * NB, D), jnp.float32),
        pltpu.VMEM((SLEN[0],), jnp.int32),
        pltpu.VMEM((SLEN[1],), jnp.int32),
        pltpu.SemaphoreType.DMA,
        pltpu.SemaphoreType.DMA,
    ],
)
def _emb_sum(table_hbm, idx_hbm, out_hbm, idx_v, rows0, rows1, out_all,
             zbuf, sacc, didx0, didx1, sem0, sem1):
    _emb_sum_body(table_hbm, idx_hbm, out_hbm, idx_v, rows0, rows1, out_all,
                  zbuf, sacc, didx0, didx1, sem0, sem1)


def _mlp_body(sums_ref, idx_ref, w1_ref, b1_ref, w2_ref, b2_ref, out_ref):
    s = sums_ref[...]
    idxb = idx_ref[...]
    cnt = jnp.sum((idxb != 0).astype(jnp.float32), axis=1, keepdims=True)
    mean = s * (1.0 / jnp.maximum(cnt, 1.0))
    h = lax.dot_general(
        mean, w1_ref[...], (((1,), (1,)), ((), ())),
        preferred_element_type=jnp.float32,
    ) + b1_ref[...]
    h = jnp.maximum(h, 0.0)
    out = lax.dot_general(
        h, w2_ref[...], (((1,), (1,)), ((), ())),
        preferred_element_type=jnp.float32,
    ) + b2_ref[...]
    out_ref[...] = out


def kernel(input_features_b_l, input_length_b, table, W1, b1, W2, b2):
    del input_length_b
    idx = input_features_b_l.astype(jnp.int32)
    idx_flat = idx.reshape(-1)
    sums = _emb_sum(table, idx_flat)
    out = pl.pallas_call(
        _mlp_body,
        out_shape=jax.ShapeDtypeStruct((B, C), jnp.float32),
    )(sums, idx, W1, b1.reshape(1, H), W2, b2.reshape(1, C))
    return out
